# initial kernel scaffold (unmeasured)
import jax
import jax.numpy as jnp
from jax import lax
from jax.experimental import pallas as pl
from jax.experimental.pallas import tpu as pltpu


def kernel(
    x,
):
    def body(*refs):
        pass

    out_shape = jax.ShapeDtypeStruct(..., jnp.float32)
    return pl.pallas_call(body, out_shape=out_shape)(...)


# baseline (device time: 890701 ns/iter reference)
import jax
import jax.numpy as jnp
from jax import lax
from jax.experimental import pallas as pl
from jax.experimental.pallas import tpu as pltpu

M = 32768
N = 1024
CH = 2048
K = M // CH


def kernel(x):
    def body(x_hbm, out_hbm, xf32, my16, rcv, res, load_sem, store_sem,
             send_sems, recv_sems):
        my_x = lax.axis_index("x")
        my_y = lax.axis_index("y")
        peer_y = (my_x, 1 - my_y)

        barrier_sem = pltpu.get_barrier_semaphore()
        pl.semaphore_signal(barrier_sem, inc=1, device_id=peer_y,
                            device_id_type=pl.DeviceIdType.MESH)
        pl.semaphore_wait(barrier_sem, 1)

        for k in range(K):
            slot = k % 2
            rows = pl.ds(k * CH, CH)

            load = pltpu.make_async_copy(x_hbm.at[rows], xf32, load_sem)
            load.start()
            load.wait()
            my16[slot] = xf32[...].astype(jnp.bfloat16)

            rdma = pltpu.make_async_remote_copy(
                src_ref=my16.at[slot],
                dst_ref=rcv.at[slot],
                send_sem=send_sems.at[slot],
                recv_sem=recv_sems.at[slot],
                device_id=peer_y,
                device_id_type=pl.DeviceIdType.MESH,
            )
            rdma.start()
            rdma.wait()

            res[...] = my16[slot] + rcv[slot]
            store = pltpu.make_async_copy(res, out_hbm.at[rows], store_sem)
            store.start()
            store.wait()

    return pl.pallas_call(
        body,
        out_shape=jax.ShapeDtypeStruct((M, N), jnp.bfloat16),
        in_specs=[pl.BlockSpec(memory_space=pl.ANY)],
        out_specs=pl.BlockSpec(memory_space=pl.ANY),
        scratch_shapes=[
            pltpu.VMEM((CH, N), jnp.float32),
            pltpu.VMEM((2, CH, N), jnp.bfloat16),
            pltpu.VMEM((2, CH, N), jnp.bfloat16),
            pltpu.VMEM((CH, N), jnp.bfloat16),
            pltpu.SemaphoreType.DMA,
            pltpu.SemaphoreType.DMA,
            pltpu.SemaphoreType.DMA((2,)),
            pltpu.SemaphoreType.DMA((2,)),
        ],
        compiler_params=pltpu.CompilerParams(collective_id=0),
    )(x)


# device time: 433279 ns/iter; 2.0557x vs baseline; 2.0557x over previous
import jax
import jax.numpy as jnp
from jax import lax
from jax.experimental import pallas as pl
from jax.experimental.pallas import tpu as pltpu

M = 32768
N = 1024
HALF = M // 2
CH = 1024
K = HALF // CH
S = 4


def kernel(x):
    def body(x_hbm, out_hbm, f32buf, my16, p1rcv, acc,
             ld_sems, st_sems, p1s, p1r, p2s, p2r):
        my_x = lax.axis_index("x")
        my_y = lax.axis_index("y")
        peer_y = (my_x, 1 - my_y)
        peer_x = (1 - my_x, my_y)

        def mine(k):
            return pl.ds(my_x * HALF + k * CH, CH)

        def theirs(k):
            return pl.ds((1 - my_x) * HALF + k * CH, CH)

        def load(j):
            return pltpu.make_async_copy(
                x_hbm.at[mine(j)], f32buf.at[j % 2], ld_sems.at[j % 2])

        def store(j):
            return pltpu.make_async_copy(
                acc.at[j % S], out_hbm.at[mine(j)], st_sems.at[j % S])

        def p1(j):
            return pltpu.make_async_remote_copy(
                src_ref=my16.at[j % S], dst_ref=p1rcv.at[j % S],
                send_sem=p1s.at[j % S], recv_sem=p1r.at[j % S],
                device_id=peer_y, device_id_type=pl.DeviceIdType.MESH)

        def p2(j):
            return pltpu.make_async_remote_copy(
                src_ref=acc.at[j % S], dst_ref=out_hbm.at[mine(j)],
                send_sem=p2s.at[j % S], recv_sem=p2r.at[j % S],
                device_id=peer_x, device_id_type=pl.DeviceIdType.MESH)

        def p2_recv(j):
            return pltpu.make_async_remote_copy(
                src_ref=acc.at[j % S], dst_ref=out_hbm.at[theirs(j)],
                send_sem=p2s.at[j % S], recv_sem=p2r.at[j % S],
                device_id=peer_x, device_id_type=pl.DeviceIdType.MESH)

        barrier_sem = pltpu.get_barrier_semaphore()
        for nbr in (peer_y, peer_x):
            pl.semaphore_signal(barrier_sem, inc=1, device_id=nbr,
                                device_id_type=pl.DeviceIdType.MESH)
        pl.semaphore_wait(barrier_sem, 2)

        load(0).start()
        load(1).start()
        load(0).wait()
        my16[0] = f32buf[0].astype(jnp.bfloat16)
        p1(0).start()
        load(1).wait()
        my16[1] = f32buf[1].astype(jnp.bfloat16)
        p1(1).start()

        for k in range(K):
            s = k % S
            if k + 2 < K:
                load(k + 2).start()
            p1(k).wait_recv()
            if k >= S:
                p2(k - S).wait_send()
                store(k - S).wait()
            acc[s] = my16[s] + p1rcv[s]
            p2(k).start()
            store(k).start()
            if k + 2 < K:
                load(k + 2).wait()
                if k >= 2:
                    p1(k - 2).wait_send()
                my16[(k + 2) % S] = f32buf[k % 2].astype(jnp.bfloat16)
                p1(k + 2).start()
            if k >= 1:
                p2_recv(k - 1).wait_recv()

        p2_recv(K - 1).wait_recv()
        for j in range(K - S, K):
            p2(j).wait_send()
            store(j).wait()
            p1(j).wait_send()

    return pl.pallas_call(
        body,
        out_shape=jax.ShapeDtypeStruct((M, N), jnp.bfloat16),
        in_specs=[pl.BlockSpec(memory_space=pl.ANY)],
        out_specs=pl.BlockSpec(memory_space=pl.ANY),
        scratch_shapes=[
            pltpu.VMEM((2, CH, N), jnp.float32),
            pltpu.VMEM((S, CH, N), jnp.bfloat16),
            pltpu.VMEM((S, CH, N), jnp.bfloat16),
            pltpu.VMEM((S, CH, N), jnp.bfloat16),
            pltpu.SemaphoreType.DMA((2,)),
            pltpu.SemaphoreType.DMA((S,)),
            pltpu.SemaphoreType.DMA((S,)),
            pltpu.SemaphoreType.DMA((S,)),
            pltpu.SemaphoreType.DMA((S,)),
            pltpu.SemaphoreType.DMA((S,)),
        ],
        compiler_params=pltpu.CompilerParams(collective_id=0),
    )(x)


# device time: 432164 ns/iter; 2.0610x vs baseline; 1.0026x over previous
import jax
import jax.numpy as jnp
from jax import lax
from jax.experimental import pallas as pl
from jax.experimental.pallas import tpu as pltpu

M = 32768
N = 1024
HALF = M // 2
CHMAX = 1024
S = 6
LOOK = 3

SIZES = [256, 256, 512] + [1024] * 14 + [512, 256, 256]
assert sum(SIZES) == HALF
OFFS = [sum(SIZES[:i]) for i in range(len(SIZES))]
K = len(SIZES)


def kernel(x):
    def body(x_hbm, out_hbm, f32buf, my16, p1rcv, acc,
             ld_sems, st_sems, p1s, p1r, p2s, p2r):
        my_x = lax.axis_index("x")
        my_y = lax.axis_index("y")
        peer_y = (my_x, 1 - my_y)
        peer_x = (1 - my_x, my_y)

        def mine(j):
            return pl.ds(my_x * HALF + OFFS[j], SIZES[j])

        def theirs(j):
            return pl.ds((1 - my_x) * HALF + OFFS[j], SIZES[j])

        def load(j):
            return pltpu.make_async_copy(
                x_hbm.at[mine(j)],
                f32buf.at[j % 2, pl.ds(0, SIZES[j])],
                ld_sems.at[j % 2])

        def store(j):
            return pltpu.make_async_copy(
                acc.at[j % S, pl.ds(0, SIZES[j])],
                out_hbm.at[mine(j)],
                st_sems.at[j % S])

        def p1(j):
            return pltpu.make_async_remote_copy(
                src_ref=my16.at[j % S, pl.ds(0, SIZES[j])],
                dst_ref=p1rcv.at[j % S, pl.ds(0, SIZES[j])],
                send_sem=p1s.at[j % S], recv_sem=p1r.at[j % S],
                device_id=peer_y, device_id_type=pl.DeviceIdType.MESH)

        def p2(j):
            return pltpu.make_async_remote_copy(
                src_ref=acc.at[j % S, pl.ds(0, SIZES[j])],
                dst_ref=out_hbm.at[mine(j)],
                send_sem=p2s.at[j % S], recv_sem=p2r.at[j % S],
                device_id=peer_x, device_id_type=pl.DeviceIdType.MESH)

        def p2_recv(j):
            return pltpu.make_async_remote_copy(
                src_ref=acc.at[j % S, pl.ds(0, SIZES[j])],
                dst_ref=out_hbm.at[theirs(j)],
                send_sem=p2s.at[j % S], recv_sem=p2r.at[j % S],
                device_id=peer_x, device_id_type=pl.DeviceIdType.MESH)

        def cast(j):
            sz = SIZES[j]
            my16[j % S, :sz] = f32buf[j % 2, :sz].astype(jnp.bfloat16)

        barrier_sem = pltpu.get_barrier_semaphore()
        for nbr in (peer_y, peer_x):
            pl.semaphore_signal(barrier_sem, inc=1, device_id=nbr,
                                device_id_type=pl.DeviceIdType.MESH)
        pl.semaphore_wait(barrier_sem, 2)

        load(0).start()
        load(1).start()
        for j in range(LOOK):
            load(j).wait()
            cast(j)
            p1(j).start()
            if j + 2 < K:
                load(j + 2).start()

        for k in range(K):
            s = k % S
            sz = SIZES[k]
            p1(k).wait_recv()
            if k >= S:
                p2(k - S).wait_send()
                store(k - S).wait()
            acc[s, :sz] = my16[s, :sz] + p1rcv[s, :sz]
            p2(k).start()
            store(k).start()
            if k + LOOK < K:
                load(k + LOOK).wait()
                if k + LOOK - S >= 0:
                    p1(k + LOOK - S).wait_send()
                cast(k + LOOK)
                p1(k + LOOK).start()
                if k + LOOK + 2 < K:
                    load(k + LOOK + 2).start()
            if k >= 1:
                p2_recv(k - 1).wait_recv()

        p2_recv(K - 1).wait_recv()
        for j in range(K - S, K):
            p2(j).wait_send()
            store(j).wait()
            p1(j).wait_send()

    return pl.pallas_call(
        body,
        out_shape=jax.ShapeDtypeStruct((M, N), jnp.bfloat16),
        in_specs=[pl.BlockSpec(memory_space=pl.ANY)],
        out_specs=pl.BlockSpec(memory_space=pl.ANY),
        scratch_shapes=[
            pltpu.VMEM((2, CHMAX, N), jnp.float32),
            pltpu.VMEM((S, CHMAX, N), jnp.bfloat16),
            pltpu.VMEM((S, CHMAX, N), jnp.bfloat16),
            pltpu.VMEM((S, CHMAX, N), jnp.bfloat16),
            pltpu.SemaphoreType.DMA((2,)),
            pltpu.SemaphoreType.DMA((S,)),
            pltpu.SemaphoreType.DMA((S,)),
            pltpu.SemaphoreType.DMA((S,)),
            pltpu.SemaphoreType.DMA((S,)),
            pltpu.SemaphoreType.DMA((S,)),
        ],
        compiler_params=pltpu.CompilerParams(
            collective_id=0, vmem_limit_bytes=64 * 1024 * 1024),
    )(x)


# device time: 420866 ns/iter; 2.1164x vs baseline; 1.0268x over previous
import jax
import jax.numpy as jnp
from jax import lax
from jax.experimental import pallas as pl
from jax.experimental.pallas import tpu as pltpu

M = 32768
N = 1024
HALF = M // 2
CHMAX = 512
S = 6
LOOK = 3

SIZES = [256, 256] + [512] * 30 + [256, 256]
assert sum(SIZES) == HALF
OFFS = [sum(SIZES[:i]) for i in range(len(SIZES))]
K = len(SIZES)


def kernel(x):
    def body(x_hbm, out_hbm, f32buf, my16, p1rcv, acc,
             ld_sems, st_sems, p1s, p1r, p2s, p2r):
        my_x = lax.axis_index("x")
        my_y = lax.axis_index("y")
        peer_y = (my_x, 1 - my_y)
        peer_x = (1 - my_x, my_y)

        def mine(j):
            return pl.ds(my_x * HALF + OFFS[j], SIZES[j])

        def theirs(j):
            return pl.ds((1 - my_x) * HALF + OFFS[j], SIZES[j])

        def load(j):
            return pltpu.make_async_copy(
                x_hbm.at[mine(j)],
                f32buf.at[j % 2, pl.ds(0, SIZES[j])],
                ld_sems.at[j % 2])

        def store(j):
            return pltpu.make_async_copy(
                acc.at[j % S, pl.ds(0, SIZES[j])],
                out_hbm.at[mine(j)],
                st_sems.at[j % S])

        def p1(j):
            return pltpu.make_async_remote_copy(
                src_ref=my16.at[j % S, pl.ds(0, SIZES[j])],
                dst_ref=p1rcv.at[j % S, pl.ds(0, SIZES[j])],
                send_sem=p1s.at[j % S], recv_sem=p1r.at[j % S],
                device_id=peer_y, device_id_type=pl.DeviceIdType.MESH)

        def p2(j):
            return pltpu.make_async_remote_copy(
                src_ref=acc.at[j % S, pl.ds(0, SIZES[j])],
                dst_ref=out_hbm.at[mine(j)],
                send_sem=p2s.at[j % S], recv_sem=p2r.at[j % S],
                device_id=peer_x, device_id_type=pl.DeviceIdType.MESH)

        def p2_recv(j):
            return pltpu.make_async_remote_copy(
                src_ref=acc.at[j % S, pl.ds(0, SIZES[j])],
                dst_ref=out_hbm.at[theirs(j)],
                send_sem=p2s.at[j % S], recv_sem=p2r.at[j % S],
                device_id=peer_x, device_id_type=pl.DeviceIdType.MESH)

        def cast(j):
            sz = SIZES[j]
            my16[j % S, :sz] = f32buf[j % 2, :sz].astype(jnp.bfloat16)

        barrier_sem = pltpu.get_barrier_semaphore()
        for nbr in (peer_y, peer_x):
            pl.semaphore_signal(barrier_sem, inc=1, device_id=nbr,
                                device_id_type=pl.DeviceIdType.MESH)
        pl.semaphore_wait(barrier_sem, 2)

        load(0).start()
        load(1).start()
        for j in range(LOOK):
            load(j).wait()
            cast(j)
            p1(j).start()
            if j + 2 < K:
                load(j + 2).start()

        for k in range(K):
            s = k % S
            sz = SIZES[k]
            p1(k).wait_recv()
            if k >= S:
                p2(k - S).wait_send()
                store(k - S).wait()
            acc[s, :sz] = my16[s, :sz] + p1rcv[s, :sz]
            p2(k).start()
            store(k).start()
            if k + LOOK < K:
                load(k + LOOK).wait()
                if k + LOOK - S >= 0:
                    p1(k + LOOK - S).wait_send()
                cast(k + LOOK)
                p1(k + LOOK).start()
                if k + LOOK + 2 < K:
                    load(k + LOOK + 2).start()
            if k >= 1:
                p2_recv(k - 1).wait_recv()

        p2_recv(K - 1).wait_recv()
        for j in range(K - S, K):
            p2(j).wait_send()
            store(j).wait()
            p1(j).wait_send()

    return pl.pallas_call(
        body,
        out_shape=jax.ShapeDtypeStruct((M, N), jnp.bfloat16),
        in_specs=[pl.BlockSpec(memory_space=pl.ANY)],
        out_specs=pl.BlockSpec(memory_space=pl.ANY),
        scratch_shapes=[
            pltpu.VMEM((2, CHMAX, N), jnp.float32),
            pltpu.VMEM((S, CHMAX, N), jnp.bfloat16),
            pltpu.VMEM((S, CHMAX, N), jnp.bfloat16),
            pltpu.VMEM((S, CHMAX, N), jnp.bfloat16),
            pltpu.SemaphoreType.DMA((2,)),
            pltpu.SemaphoreType.DMA((S,)),
            pltpu.SemaphoreType.DMA((S,)),
            pltpu.SemaphoreType.DMA((S,)),
            pltpu.SemaphoreType.DMA((S,)),
            pltpu.SemaphoreType.DMA((S,)),
        ],
        compiler_params=pltpu.CompilerParams(
            collective_id=0, vmem_limit_bytes=64 * 1024 * 1024),
    )(x)


# device time: 415218 ns/iter; 2.1451x vs baseline; 1.0136x over previous
import jax
import jax.numpy as jnp
from jax import lax
from jax.experimental import pallas as pl
from jax.experimental.pallas import tpu as pltpu

M = 32768
N = 1024
HALF = M // 2
CHMAX = 256
S = 6
LOOK = 3

SIZES = [256] * 64
assert sum(SIZES) == HALF
OFFS = [sum(SIZES[:i]) for i in range(len(SIZES))]
K = len(SIZES)


def kernel(x):
    def body(x_hbm, out_hbm, f32buf, my16, p1rcv, acc,
             ld_sems, st_sems, p1s, p1r, p2s, p2r):
        my_x = lax.axis_index("x")
        my_y = lax.axis_index("y")
        peer_y = (my_x, 1 - my_y)
        peer_x = (1 - my_x, my_y)

        def mine(j):
            return pl.ds(my_x * HALF + OFFS[j], SIZES[j])

        def theirs(j):
            return pl.ds((1 - my_x) * HALF + OFFS[j], SIZES[j])

        def load(j):
            return pltpu.make_async_copy(
                x_hbm.at[mine(j)],
                f32buf.at[j % 2, pl.ds(0, SIZES[j])],
                ld_sems.at[j % 2])

        def store(j):
            return pltpu.make_async_copy(
                acc.at[j % S, pl.ds(0, SIZES[j])],
                out_hbm.at[mine(j)],
                st_sems.at[j % S])

        def p1(j):
            return pltpu.make_async_remote_copy(
                src_ref=my16.at[j % S, pl.ds(0, SIZES[j])],
                dst_ref=p1rcv.at[j % S, pl.ds(0, SIZES[j])],
                send_sem=p1s.at[j % S], recv_sem=p1r.at[j % S],
                device_id=peer_y, device_id_type=pl.DeviceIdType.MESH)

        def p2(j):
            return pltpu.make_async_remote_copy(
                src_ref=acc.at[j % S, pl.ds(0, SIZES[j])],
                dst_ref=out_hbm.at[mine(j)],
                send_sem=p2s.at[j % S], recv_sem=p2r.at[j % S],
                device_id=peer_x, device_id_type=pl.DeviceIdType.MESH)

        def p2_recv(j):
            return pltpu.make_async_remote_copy(
                src_ref=acc.at[j % S, pl.ds(0, SIZES[j])],
                dst_ref=out_hbm.at[theirs(j)],
                send_sem=p2s.at[j % S], recv_sem=p2r.at[j % S],
                device_id=peer_x, device_id_type=pl.DeviceIdType.MESH)

        def cast(j):
            sz = SIZES[j]
            my16[j % S, :sz] = f32buf[j % 2, :sz].astype(jnp.bfloat16)

        barrier_sem = pltpu.get_barrier_semaphore()
        for nbr in (peer_y, peer_x):
            pl.semaphore_signal(barrier_sem, inc=1, device_id=nbr,
                                device_id_type=pl.DeviceIdType.MESH)
        pl.semaphore_wait(barrier_sem, 2)

        load(0).start()
        load(1).start()
        for j in range(LOOK):
            load(j).wait()
            cast(j)
            p1(j).start()
            if j + 2 < K:
                load(j + 2).start()

        for k in range(K):
            s = k % S
            sz = SIZES[k]
            p1(k).wait_recv()
            if k >= S:
                p2(k - S).wait_send()
                store(k - S).wait()
            acc[s, :sz] = my16[s, :sz] + p1rcv[s, :sz]
            p2(k).start()
            store(k).start()
            if k + LOOK < K:
                load(k + LOOK).wait()
                if k + LOOK - S >= 0:
                    p1(k + LOOK - S).wait_send()
                cast(k + LOOK)
                p1(k + LOOK).start()
                if k + LOOK + 2 < K:
                    load(k + LOOK + 2).start()
            if k >= 1:
                p2_recv(k - 1).wait_recv()

        p2_recv(K - 1).wait_recv()
        for j in range(K - S, K):
            p2(j).wait_send()
            store(j).wait()
            p1(j).wait_send()

    return pl.pallas_call(
        body,
        out_shape=jax.ShapeDtypeStruct((M, N), jnp.bfloat16),
        in_specs=[pl.BlockSpec(memory_space=pl.ANY)],
        out_specs=pl.BlockSpec(memory_space=pl.ANY),
        scratch_shapes=[
            pltpu.VMEM((2, CHMAX, N), jnp.float32),
            pltpu.VMEM((S, CHMAX, N), jnp.bfloat16),
            pltpu.VMEM((S, CHMAX, N), jnp.bfloat16),
            pltpu.VMEM((S, CHMAX, N), jnp.bfloat16),
            pltpu.SemaphoreType.DMA((2,)),
            pltpu.SemaphoreType.DMA((S,)),
            pltpu.SemaphoreType.DMA((S,)),
            pltpu.SemaphoreType.DMA((S,)),
            pltpu.SemaphoreType.DMA((S,)),
            pltpu.SemaphoreType.DMA((S,)),
        ],
        compiler_params=pltpu.CompilerParams(
            collective_id=0, vmem_limit_bytes=64 * 1024 * 1024),
    )(x)
